# baseline (speedup 1.0000x reference)
import functools

import jax
import jax.numpy as jnp
from jax import lax
from jax.experimental import pallas as pl
from jax.experimental.pallas import tpu as pltpu
from jax.experimental.pallas import tpu_sc as plsc

F32 = jnp.float32

N = 10000
D = 128
E = 320000
NC, NS = 2, 16
NW = NC * NS
CHUNK = 128
CPT = 80
EPT = CPT * CHUNK
EPAD = NW * EPT
NPAD = 10240
NPT = NPAD // NS
ROWS_BLK = 1000

_mesh = plsc.VectorSubcoreMesh(
    core_axis_name="c", subcore_axis_name="s", num_cores=NC, num_subcores=NS
)


def _deg_body(dstb_hbm, zeros1_hbm, degp_hbm, idx_v, ones_v, vb, acc_sh):
    c = lax.axis_index("c")
    s = lax.axis_index("s")
    tile = c * NS + s

    pltpu.sync_copy(zeros1_hbm.at[pl.ds(s * NPT, NPT)], vb)
    pltpu.sync_copy(vb, acc_sh.at[pl.ds(s * NPT, NPT)])

    for j in range(CHUNK // 16):
        ones_v[pl.ds(j * 16, 16)] = jnp.ones((16,), F32)
    pltpu.sync_copy(dstb_hbm.at[tile], idx_v)
    plsc.subcore_barrier()

    def body(i, carry):
        pltpu.sync_copy(ones_v, acc_sh.at[idx_v.at[i]], add=True)
        return carry

    lax.fori_loop(0, CPT, body, 0)
    plsc.subcore_barrier()

    pltpu.sync_copy(acc_sh.at[pl.ds(s * NPT, NPT)], vb)
    pltpu.sync_copy(vb, degp_hbm.at[pl.ds(c * NPAD + s * NPT, NPT)])


_deg_call = functools.partial(
    pl.kernel,
    out_type=jax.ShapeDtypeStruct((NC * NPAD,), F32),
    mesh=_mesh,
    scratch_types=[
        pltpu.VMEM((CPT, CHUNK), jnp.int32),
        pltpu.VMEM((CHUNK,), F32),
        pltpu.VMEM((NPT,), F32),
        pltpu.VMEM_SHARED((NPAD,), F32),
    ],
)(_deg_body)


def _scat_body(h_hbm, srcb_hbm, dstb_hbm, zeros2_hbm, out_hbm,
               idxs_v, idxd_v, rows_v, acc_sh, sem):
    c = lax.axis_index("c")
    s = lax.axis_index("s")
    tile = c * NS + s

    pltpu.sync_copy(zeros2_hbm, rows_v)
    for k in range(NPT // CHUNK):
        pltpu.sync_copy(rows_v, acc_sh.at[pl.ds((s * (NPT // CHUNK) + k) * CHUNK, CHUNK)])

    pltpu.sync_copy(srcb_hbm.at[tile], idxs_v)
    pltpu.sync_copy(dstb_hbm.at[tile], idxd_v)
    plsc.subcore_barrier()

    def body(i, carry):
        pltpu.async_copy(h_hbm.at[idxs_v.at[i]], rows_v, sem).wait()
        pltpu.sync_copy(rows_v, acc_sh.at[idxd_v.at[i]], add=True)
        return carry

    lax.fori_loop(0, CPT, body, 0)
    plsc.subcore_barrier()

    for k in range(NPT // CHUNK):
        base = (s * (NPT // CHUNK) + k) * CHUNK
        pltpu.sync_copy(acc_sh.at[pl.ds(base, CHUNK)], rows_v)
        pltpu.sync_copy(rows_v, out_hbm.at[c].at[pl.ds(base, CHUNK)])


_scat_call = functools.partial(
    pl.kernel,
    out_type=jax.ShapeDtypeStruct((NC, NPAD, D), F32),
    mesh=_mesh,
    scratch_types=[
        pltpu.VMEM((CPT, CHUNK), jnp.int32),
        pltpu.VMEM((CPT, CHUNK), jnp.int32),
        pltpu.VMEM((CHUNK, D), F32),
        pltpu.VMEM_SHARED((NPAD, D), F32),
        pltpu.SemaphoreType.DMA,
    ],
)(_scat_body)


def _pre_body(degt_ref, x_ref, w_ref, dinv_ref, hp_ref):
    deg = degt_ref[:, 0:1] + degt_ref[:, 1:2] + 1.0
    dinv = lax.rsqrt(deg)
    dinv_ref[...] = dinv
    hp_ref[...] = dinv * jnp.dot(
        x_ref[...], w_ref[...], preferred_element_type=F32
    )


def _mid_body(sp_ref, hp_ref, dinv_ref, b_ref, w_ref, o_ref):
    dinv = dinv_ref[...]
    x2 = dinv * (sp_ref[0] + sp_ref[1] + hp_ref[...]) + b_ref[...]
    o_ref[...] = dinv * jnp.dot(x2, w_ref[...], preferred_element_type=F32)


def _post_body(sp_ref, hp_ref, dinv_ref, b_ref, o_ref):
    o_ref[...] = (
        dinv_ref[...] * (sp_ref[0] + sp_ref[1] + hp_ref[...]) + b_ref[...]
    )


_GRID = (N // ROWS_BLK,)
_spec_rows = pl.BlockSpec((ROWS_BLK, D), lambda i: (i, 0))
_spec_degt = pl.BlockSpec((ROWS_BLK, NC), lambda i: (i, 0))
_spec_dinv = pl.BlockSpec((ROWS_BLK, 1), lambda i: (i, 0))
_spec_w = pl.BlockSpec((D, D), lambda i: (0, 0))
_spec_b = pl.BlockSpec((D,), lambda i: (0,))
_spec_sp = pl.BlockSpec((NC, ROWS_BLK, D), lambda i: (0, i, 0))


def _pre_call(degt, x, w):
    return pl.pallas_call(
        _pre_body,
        grid=_GRID,
        in_specs=[_spec_degt, _spec_rows, _spec_w],
        out_specs=[_spec_dinv, _spec_rows],
        out_shape=[
            jax.ShapeDtypeStruct((N, 1), F32),
            jax.ShapeDtypeStruct((N, D), F32),
        ],
    )(degt, x, w)


def _mid_call(sp, hp, dinv, b, w):
    return pl.pallas_call(
        _mid_body,
        grid=_GRID,
        in_specs=[_spec_sp, _spec_rows, _spec_dinv, _spec_b, _spec_w],
        out_specs=_spec_rows,
        out_shape=jax.ShapeDtypeStruct((N, D), F32),
    )(sp, hp, dinv, b, w)


def _post_call(sp, hp, dinv, b):
    return pl.pallas_call(
        _post_body,
        grid=_GRID,
        in_specs=[_spec_sp, _spec_rows, _spec_dinv, _spec_b],
        out_specs=_spec_rows,
        out_shape=jax.ShapeDtypeStruct((N, D), F32),
    )(sp, hp, dinv, b)


@jax.jit
def kernel(x, edge_index, W1, b1, W2, b2, W3, b3):
    ei = edge_index.astype(jnp.int32)
    npad = EPAD - E
    src_t = jnp.concatenate(
        [ei[0], jnp.zeros((npad,), jnp.int32)]
    ).reshape(NW, CPT, CHUNK)
    pad_dst = N + (jnp.arange(npad, dtype=jnp.int32) % (NPAD - N))
    dst_t = jnp.concatenate([ei[1], pad_dst]).reshape(NW, CPT, CHUNK)
    zeros1 = jnp.zeros((NPAD,), F32)
    zeros2 = jnp.zeros((CHUNK, D), F32)

    degt = _deg_call(dst_t, zeros1).reshape(NC, NPAD)[:, :N].T

    dinv, h1p = _pre_call(degt, x, W1)
    s1 = _scat_call(h1p, src_t, dst_t, zeros2)[:, :N]
    h2p = _mid_call(s1, h1p, dinv, b1, W2)
    s2 = _scat_call(h2p, src_t, dst_t, zeros2)[:, :N]
    h3p = _mid_call(s2, h2p, dinv, b2, W3)
    s3 = _scat_call(h3p, src_t, dst_t, zeros2)[:, :N]
    return _post_call(s3, h3p, dinv, b3)


# speedup vs baseline: 2.5072x; 2.5072x over previous
import functools

import jax
import jax.numpy as jnp
from jax import lax
from jax.experimental import pallas as pl
from jax.experimental.pallas import tpu as pltpu
from jax.experimental.pallas import tpu_sc as plsc

F32 = jnp.float32

N = 10000
D = 128
E = 320000
NC, NS = 2, 16
NW = NC * NS
CHUNK = 128
CPT = 80
EPT = CPT * CHUNK
EPAD = NW * EPT
NPAD = 10240
NPT = NPAD // NS
ROWS_BLK = 1000

_mesh = plsc.VectorSubcoreMesh(
    core_axis_name="c", subcore_axis_name="s", num_cores=NC, num_subcores=NS
)


def _deg_body(dstb_hbm, zeros1_hbm, degp_hbm, idx_v, ones_v, vb, acc_sh):
    c = lax.axis_index("c")
    s = lax.axis_index("s")
    tile = c * NS + s

    pltpu.sync_copy(zeros1_hbm.at[pl.ds(s * NPT, NPT)], vb)
    pltpu.sync_copy(vb, acc_sh.at[pl.ds(s * NPT, NPT)])

    for j in range(CHUNK // 16):
        ones_v[pl.ds(j * 16, 16)] = jnp.ones((16,), F32)
    pltpu.sync_copy(dstb_hbm.at[tile], idx_v)
    plsc.subcore_barrier()

    def body(i, carry):
        pltpu.sync_copy(ones_v, acc_sh.at[idx_v.at[i]], add=True)
        return carry

    lax.fori_loop(0, CPT, body, 0)
    plsc.subcore_barrier()

    pltpu.sync_copy(acc_sh.at[pl.ds(s * NPT, NPT)], vb)
    pltpu.sync_copy(vb, degp_hbm.at[pl.ds(c * NPAD + s * NPT, NPT)])


_deg_call = functools.partial(
    pl.kernel,
    out_type=jax.ShapeDtypeStruct((NC * NPAD,), F32),
    mesh=_mesh,
    scratch_types=[
        pltpu.VMEM((CPT, CHUNK), jnp.int32),
        pltpu.VMEM((CHUNK,), F32),
        pltpu.VMEM((NPT,), F32),
        pltpu.VMEM_SHARED((NPAD,), F32),
    ],
)(_deg_body)


def _scat_body(h_hbm, srcb_hbm, dstb_hbm, zeros2_hbm, out_hbm,
               idxs_v, idxd_v, rows_v, acc_sh, sem):
    c = lax.axis_index("c")
    s = lax.axis_index("s")
    tile = c * NS + s

    pltpu.sync_copy(zeros2_hbm, rows_v)
    for k in range(NPT // CHUNK):
        pltpu.sync_copy(rows_v, acc_sh.at[pl.ds((s * (NPT // CHUNK) + k) * CHUNK, CHUNK)])

    pltpu.sync_copy(srcb_hbm.at[tile], idxs_v)
    pltpu.sync_copy(dstb_hbm.at[tile], idxd_v)
    plsc.subcore_barrier()

    def body(i, carry):
        pltpu.async_copy(h_hbm.at[idxs_v.at[i]], rows_v, sem).wait()
        pltpu.sync_copy(rows_v, acc_sh.at[idxd_v.at[i]], add=True)
        return carry

    lax.fori_loop(0, CPT, body, 0)
    plsc.subcore_barrier()

    for k in range(NPT // CHUNK):
        base = (s * (NPT // CHUNK) + k) * CHUNK
        pltpu.sync_copy(acc_sh.at[pl.ds(base, CHUNK)], rows_v)
        pltpu.sync_copy(rows_v, out_hbm.at[c].at[pl.ds(base, CHUNK)])


_scat_call = functools.partial(
    pl.kernel,
    out_type=jax.ShapeDtypeStruct((NC, NPAD, D), F32),
    mesh=_mesh,
    scratch_types=[
        pltpu.VMEM((CPT, CHUNK), jnp.int32),
        pltpu.VMEM((CPT, CHUNK), jnp.int32),
        pltpu.VMEM((CHUNK, D), F32),
        pltpu.VMEM_SHARED((NPAD, D), F32),
        pltpu.SemaphoreType.DMA,
    ],
)(_scat_body)


def _pre_body(degt_ref, x_ref, w_ref, dinv_ref, hp_ref):
    deg = degt_ref[:, 0:1] + degt_ref[:, 1:2] + 1.0
    dinv = lax.rsqrt(deg)
    dinv_ref[...] = dinv
    hp_ref[...] = dinv * jnp.dot(
        x_ref[...], w_ref[...], preferred_element_type=F32
    )


def _mid_body(sp_ref, hp_ref, dinv_ref, b_ref, w_ref, o_ref):
    dinv = dinv_ref[...]
    x2 = dinv * (sp_ref[0] + sp_ref[1] + hp_ref[...]) + b_ref[...]
    o_ref[...] = dinv * jnp.dot(x2, w_ref[...], preferred_element_type=F32)


def _post_body(sp_ref, hp_ref, dinv_ref, b_ref, o_ref):
    o_ref[...] = (
        dinv_ref[...] * (sp_ref[0] + sp_ref[1] + hp_ref[...]) + b_ref[...]
    )


_GRID = (N // ROWS_BLK,)
_spec_rows = pl.BlockSpec((ROWS_BLK, D), lambda i: (i, 0))
_spec_degt = pl.BlockSpec((ROWS_BLK, NC), lambda i: (i, 0))
_spec_dinv = pl.BlockSpec((ROWS_BLK, 1), lambda i: (i, 0))
_spec_w = pl.BlockSpec((D, D), lambda i: (0, 0))
_spec_b = pl.BlockSpec((D,), lambda i: (0,))
_spec_sp = pl.BlockSpec((NC, ROWS_BLK, D), lambda i: (0, i, 0))


def _pre_call(degt, x, w):
    return pl.pallas_call(
        _pre_body,
        grid=_GRID,
        in_specs=[_spec_degt, _spec_rows, _spec_w],
        out_specs=[_spec_dinv, _spec_rows],
        out_shape=[
            jax.ShapeDtypeStruct((N, 1), F32),
            jax.ShapeDtypeStruct((N, D), F32),
        ],
    )(degt, x, w)


def _mid_call(sp, hp, dinv, b, w):
    return pl.pallas_call(
        _mid_body,
        grid=_GRID,
        in_specs=[_spec_sp, _spec_rows, _spec_dinv, _spec_b, _spec_w],
        out_specs=_spec_rows,
        out_shape=jax.ShapeDtypeStruct((N, D), F32),
    )(sp, hp, dinv, b, w)


def _post_call(sp, hp, dinv, b):
    return pl.pallas_call(
        _post_body,
        grid=_GRID,
        in_specs=[_spec_sp, _spec_rows, _spec_dinv, _spec_b],
        out_specs=_spec_rows,
        out_shape=jax.ShapeDtypeStruct((N, D), F32),
    )(sp, hp, dinv, b)


@jax.jit
def kernel(x, edge_index, W1, b1, W2, b2, W3, b3):
    ei = edge_index.astype(jnp.int32)
    npad = EPAD - E
    pad_src = jnp.arange(npad, dtype=jnp.int32) % N
    src_t = jnp.concatenate([ei[0], pad_src]).reshape(NW, CPT, CHUNK)
    pad_dst = N + (jnp.arange(npad, dtype=jnp.int32) % (NPAD - N))
    dst_t = jnp.concatenate([ei[1], pad_dst]).reshape(NW, CPT, CHUNK)
    zeros1 = jnp.zeros((NPAD,), F32)
    zeros2 = jnp.zeros((CHUNK, D), F32)

    degt = _deg_call(dst_t, zeros1).reshape(NC, NPAD)[:, :N].T

    dinv, h1p = _pre_call(degt, x, W1)
    s1 = _scat_call(h1p, src_t, dst_t, zeros2)[:, :N]
    h2p = _mid_call(s1, h1p, dinv, b1, W2)
    s2 = _scat_call(h2p, src_t, dst_t, zeros2)[:, :N]
    h3p = _mid_call(s2, h2p, dinv, b2, W3)
    s3 = _scat_call(h3p, src_t, dst_t, zeros2)[:, :N]
    return _post_call(s3, h3p, dinv, b3)


# speedup vs baseline: 3.6685x; 1.4632x over previous
import functools

import jax
import jax.numpy as jnp
from jax import lax
from jax.experimental import pallas as pl
from jax.experimental.pallas import tpu as pltpu
from jax.experimental.pallas import tpu_sc as plsc

F32 = jnp.float32

N = 10000
D = 128
E = 320000
NC, NS = 2, 16
NW = NC * NS
CHUNK = 128
CPT = 80
BLKC = 8
NBLK = CPT // BLKC
EPT = CPT * CHUNK
EPAD = NW * EPT
NPAD = 10240
NPT = NPAD // NS
ROWS_BLK = 1000

_mesh = plsc.VectorSubcoreMesh(
    core_axis_name="c", subcore_axis_name="s", num_cores=NC, num_subcores=NS
)


def _deg_body(dstb_hbm, zeros1_hbm, degp_hbm, idx_v, ones_v, vb, acc_sh):
    c = lax.axis_index("c")
    s = lax.axis_index("s")
    tile = c * NS + s

    pltpu.sync_copy(zeros1_hbm.at[pl.ds(s * NPT, NPT)], vb)
    pltpu.sync_copy(vb, acc_sh.at[pl.ds(s * NPT, NPT)])

    for j in range(CHUNK // 16):
        ones_v[pl.ds(j * 16, 16)] = jnp.ones((16,), F32)
    plsc.subcore_barrier()

    def blk_body(blk, carry):
        pltpu.sync_copy(dstb_hbm.at[tile].at[pl.ds(blk * 16, 16)], idx_v)

        def body(i, c2):
            pltpu.sync_copy(ones_v, acc_sh.at[idx_v.at[i]], add=True)
            return c2

        return lax.fori_loop(0, 16, body, carry)

    lax.fori_loop(0, CPT // 16, blk_body, 0)
    plsc.subcore_barrier()

    pltpu.sync_copy(acc_sh.at[pl.ds(s * NPT, NPT)], vb)
    pltpu.sync_copy(vb, degp_hbm.at[pl.ds(c * NPAD + s * NPT, NPT)])


_deg_call = functools.partial(
    pl.kernel,
    out_type=jax.ShapeDtypeStruct((NC * NPAD,), F32),
    mesh=_mesh,
    scratch_types=[
        pltpu.VMEM((16, CHUNK), jnp.int32),
        pltpu.VMEM((CHUNK,), F32),
        pltpu.VMEM((NPT,), F32),
        pltpu.VMEM_SHARED((NPAD,), F32),
    ],
)(_deg_body)


def _scat_body(h_hbm, srcb_hbm, dstb_hbm, zeros2_hbm, out_hbm,
               idxs_v, idxd0_v, idxd1_v, rows0_v, rows1_v, acc_sh,
               gsem0, gsem1, isem0, isem1):
    c = lax.axis_index("c")
    s = lax.axis_index("s")
    tile = c * NS + s
    rows = (rows0_v, rows1_v)
    gsems = (gsem0, gsem1)
    idxd = (idxd0_v, idxd1_v)
    isems = (isem0, isem1)

    pltpu.sync_copy(srcb_hbm.at[tile], idxs_v)

    pltpu.sync_copy(zeros2_hbm, rows0_v)
    for k in range(NPT // CHUNK):
        pltpu.sync_copy(rows0_v, acc_sh.at[pl.ds((s * (NPT // CHUNK) + k) * CHUNK, CHUNK)])

    for b in range(2):
        pltpu.async_copy(dstb_hbm.at[tile].at[pl.ds(b * BLKC, BLKC)], idxd[b], isems[b])
        pltpu.async_copy(h_hbm.at[idxs_v.at[b]], rows[b], gsems[b])
    plsc.subcore_barrier()

    def chunk_pair(blk, ob, kk):
        for b in range(2):
            j = blk * BLKC + 2 * kk + b
            pltpu.make_async_copy(h_hbm.at[idxs_v.at[j]], rows[b], gsems[b]).wait()
            pltpu.sync_copy(rows[b], acc_sh.at[idxd[ob].at[2 * kk + b]], add=True)

            @pl.when(j + 2 < CPT)
            def _next_gather():
                pltpu.async_copy(h_hbm.at[idxs_v.at[j + 2]], rows[b], gsems[b])

    def outer(o, carry):
        for ob in range(2):
            blk = 2 * o + ob
            pltpu.make_async_copy(
                dstb_hbm.at[tile].at[pl.ds(blk * BLKC, BLKC)], idxd[ob], isems[ob]
            ).wait()

            def inner(kk, c2):
                chunk_pair(blk, ob, kk)
                return c2

            lax.fori_loop(0, BLKC // 2, inner, 0)

            @pl.when(blk + 2 < NBLK)
            def _next_idx_blk():
                pltpu.async_copy(
                    dstb_hbm.at[tile].at[pl.ds((blk + 2) * BLKC, BLKC)],
                    idxd[ob],
                    isems[ob],
                )

        return carry

    lax.fori_loop(0, NBLK // 2, outer, 0)
    plsc.subcore_barrier()

    for k in range(NPT // CHUNK):
        base = (s * (NPT // CHUNK) + k) * CHUNK
        pltpu.sync_copy(acc_sh.at[pl.ds(base, CHUNK)], rows0_v)
        pltpu.sync_copy(rows0_v, out_hbm.at[c].at[pl.ds(base, CHUNK)])


_scat_call = functools.partial(
    pl.kernel,
    out_type=jax.ShapeDtypeStruct((NC, NPAD, D), F32),
    mesh=_mesh,
    scratch_types=[
        pltpu.VMEM((CPT, CHUNK), jnp.int32),
        pltpu.VMEM((BLKC, CHUNK), jnp.int32),
        pltpu.VMEM((BLKC, CHUNK), jnp.int32),
        pltpu.VMEM((CHUNK, D), F32),
        pltpu.VMEM((CHUNK, D), F32),
        pltpu.VMEM_SHARED((NPAD, D), F32),
        pltpu.SemaphoreType.DMA,
        pltpu.SemaphoreType.DMA,
        pltpu.SemaphoreType.DMA,
        pltpu.SemaphoreType.DMA,
    ],
)(_scat_body)


def _pre_body(degt_ref, x_ref, w_ref, dinv_ref, hp_ref):
    deg = degt_ref[:, 0:1] + degt_ref[:, 1:2] + 1.0
    dinv = lax.rsqrt(deg)
    dinv_ref[...] = dinv
    hp_ref[...] = dinv * jnp.dot(
        x_ref[...], w_ref[...], preferred_element_type=F32
    )


def _mid_body(sp_ref, hp_ref, dinv_ref, b_ref, w_ref, o_ref):
    dinv = dinv_ref[...]
    x2 = dinv * (sp_ref[0] + sp_ref[1] + hp_ref[...]) + b_ref[...]
    o_ref[...] = dinv * jnp.dot(x2, w_ref[...], preferred_element_type=F32)


def _post_body(sp_ref, hp_ref, dinv_ref, b_ref, o_ref):
    o_ref[...] = (
        dinv_ref[...] * (sp_ref[0] + sp_ref[1] + hp_ref[...]) + b_ref[...]
    )


_GRID = (N // ROWS_BLK,)
_spec_rows = pl.BlockSpec((ROWS_BLK, D), lambda i: (i, 0))
_spec_degt = pl.BlockSpec((ROWS_BLK, NC), lambda i: (i, 0))
_spec_dinv = pl.BlockSpec((ROWS_BLK, 1), lambda i: (i, 0))
_spec_w = pl.BlockSpec((D, D), lambda i: (0, 0))
_spec_b = pl.BlockSpec((D,), lambda i: (0,))
_spec_sp = pl.BlockSpec((NC, ROWS_BLK, D), lambda i: (0, i, 0))


def _pre_call(degt, x, w):
    return pl.pallas_call(
        _pre_body,
        grid=_GRID,
        in_specs=[_spec_degt, _spec_rows, _spec_w],
        out_specs=[_spec_dinv, _spec_rows],
        out_shape=[
            jax.ShapeDtypeStruct((N, 1), F32),
            jax.ShapeDtypeStruct((N, D), F32),
        ],
    )(degt, x, w)


def _mid_call(sp, hp, dinv, b, w):
    return pl.pallas_call(
        _mid_body,
        grid=_GRID,
        in_specs=[_spec_sp, _spec_rows, _spec_dinv, _spec_b, _spec_w],
        out_specs=_spec_rows,
        out_shape=jax.ShapeDtypeStruct((N, D), F32),
    )(sp, hp, dinv, b, w)


def _post_call(sp, hp, dinv, b):
    return pl.pallas_call(
        _post_body,
        grid=_GRID,
        in_specs=[_spec_sp, _spec_rows, _spec_dinv, _spec_b],
        out_specs=_spec_rows,
        out_shape=jax.ShapeDtypeStruct((N, D), F32),
    )(sp, hp, dinv, b)


@jax.jit
def kernel(x, edge_index, W1, b1, W2, b2, W3, b3):
    ei = edge_index.astype(jnp.int32)
    npad = EPAD - E
    pad_src = jnp.arange(npad, dtype=jnp.int32) % N
    src_t = jnp.concatenate([ei[0], pad_src]).reshape(NW, CPT, CHUNK)
    pad_dst = N + (jnp.arange(npad, dtype=jnp.int32) % (NPAD - N))
    dst_t = jnp.concatenate([ei[1], pad_dst]).reshape(NW, CPT, CHUNK)
    zeros1 = jnp.zeros((NPAD,), F32)
    zeros2 = jnp.zeros((CHUNK, D), F32)

    degt = _deg_call(dst_t, zeros1).reshape(NC, NPAD)[:, :N].T

    dinv, h1p = _pre_call(degt, x, W1)
    s1 = _scat_call(h1p, src_t, dst_t, zeros2)[:, :N]
    h2p = _mid_call(s1, h1p, dinv, b1, W2)
    s2 = _scat_call(h2p, src_t, dst_t, zeros2)[:, :N]
    h3p = _mid_call(s2, h2p, dinv, b2, W3)
    s3 = _scat_call(h3p, src_t, dst_t, zeros2)[:, :N]
    return _post_call(s3, h3p, dinv, b3)


# speedup vs baseline: 3.8450x; 1.0481x over previous
import functools

import jax
import jax.numpy as jnp
from jax import lax
from jax.experimental import pallas as pl
from jax.experimental.pallas import tpu as pltpu
from jax.experimental.pallas import tpu_sc as plsc

F32 = jnp.float32

N = 10000
D = 128
E = 320000
NC, NS = 2, 16
NW = NC * NS
CHUNK = 128
CPT = 80
BLKC = 8
NBLK = CPT // BLKC
EPT = CPT * CHUNK
EPAD = NW * EPT
NPAD = 10240
NPT = NPAD // NS
ROWS_BLK = 1000

_mesh = plsc.VectorSubcoreMesh(
    core_axis_name="c", subcore_axis_name="s", num_cores=NC, num_subcores=NS
)


def _deg_body(dstb_hbm, zeros1_hbm, degp_hbm, idx_v, ones_v, vb, acc_sh):
    c = lax.axis_index("c")
    s = lax.axis_index("s")
    tile = c * NS + s

    pltpu.sync_copy(zeros1_hbm.at[pl.ds(s * NPT, NPT)], vb)
    pltpu.sync_copy(vb, acc_sh.at[pl.ds(s * NPT, NPT)])

    for j in range(CHUNK // 16):
        ones_v[pl.ds(j * 16, 16)] = jnp.ones((16,), F32)
    plsc.subcore_barrier()

    def blk_body(blk, carry):
        pltpu.sync_copy(dstb_hbm.at[tile].at[pl.ds(blk * 16, 16)], idx_v)

        def body(i, c2):
            pltpu.sync_copy(ones_v, acc_sh.at[idx_v.at[i]], add=True)
            return c2

        return lax.fori_loop(0, 16, body, carry)

    lax.fori_loop(0, CPT // 16, blk_body, 0)
    plsc.subcore_barrier()

    pltpu.sync_copy(acc_sh.at[pl.ds(s * NPT, NPT)], vb)
    pltpu.sync_copy(vb, degp_hbm.at[pl.ds(c * NPAD + s * NPT, NPT)])


_deg_call = functools.partial(
    pl.kernel,
    out_type=jax.ShapeDtypeStruct((NC * NPAD,), F32),
    mesh=_mesh,
    scratch_types=[
        pltpu.VMEM((16, CHUNK), jnp.int32),
        pltpu.VMEM((CHUNK,), F32),
        pltpu.VMEM((NPT,), F32),
        pltpu.VMEM_SHARED((NPAD,), F32),
    ],
)(_deg_body)


def _scat_body(h_hbm, srcb_hbm, dstb_hbm, zeros2_hbm, out_hbm,
               idxs_v, idxd0_v, idxd1_v, rows0_v, rows1_v, acc_sh,
               gsem0, gsem1, isem0, isem1):
    c = lax.axis_index("c")
    s = lax.axis_index("s")
    tile = c * NS + s
    rows = (rows0_v, rows1_v)
    gsems = (gsem0, gsem1)
    idxd = (idxd0_v, idxd1_v)
    isems = (isem0, isem1)

    pltpu.sync_copy(srcb_hbm.at[tile], idxs_v)

    pltpu.sync_copy(zeros2_hbm, rows0_v)
    for k in range(NPT // CHUNK):
        pltpu.sync_copy(rows0_v, acc_sh.at[pl.ds((s * (NPT // CHUNK) + k) * CHUNK, CHUNK)])

    for b in range(2):
        pltpu.async_copy(dstb_hbm.at[tile].at[pl.ds(b * BLKC, BLKC)], idxd[b], isems[b])
        pltpu.async_copy(h_hbm.at[idxs_v.at[b]], rows[b], gsems[b])
    plsc.subcore_barrier()

    def chunk_pair(blk, ob, kk):
        for b in range(2):
            j = blk * BLKC + 2 * kk + b
            pltpu.make_async_copy(h_hbm.at[idxs_v.at[j]], rows[b], gsems[b]).wait()
            pltpu.sync_copy(rows[b], acc_sh.at[idxd[ob].at[2 * kk + b]], add=True)

            @pl.when(j + 2 < CPT)
            def _next_gather():
                pltpu.async_copy(h_hbm.at[idxs_v.at[j + 2]], rows[b], gsems[b])

    def outer(o, carry):
        for ob in range(2):
            blk = 2 * o + ob
            pltpu.make_async_copy(
                dstb_hbm.at[tile].at[pl.ds(blk * BLKC, BLKC)], idxd[ob], isems[ob]
            ).wait()

            def inner(kk, c2):
                chunk_pair(blk, ob, kk)
                return c2

            lax.fori_loop(0, BLKC // 2, inner, 0)

            @pl.when(blk + 2 < NBLK)
            def _next_idx_blk():
                pltpu.async_copy(
                    dstb_hbm.at[tile].at[pl.ds((blk + 2) * BLKC, BLKC)],
                    idxd[ob],
                    isems[ob],
                )

        return carry

    lax.fori_loop(0, NBLK // 2, outer, 0)
    plsc.subcore_barrier()

    for k in range(NPT // CHUNK):
        base = (s * (NPT // CHUNK) + k) * CHUNK
        pltpu.sync_copy(acc_sh.at[pl.ds(base, CHUNK)], rows0_v)
        pltpu.sync_copy(rows0_v, out_hbm.at[c].at[pl.ds(base, CHUNK)])


_scat_call = functools.partial(
    pl.kernel,
    out_type=jax.ShapeDtypeStruct((NC, NPAD, D), F32),
    mesh=_mesh,
    scratch_types=[
        pltpu.VMEM((CPT, CHUNK), jnp.int32),
        pltpu.VMEM((BLKC, CHUNK), jnp.int32),
        pltpu.VMEM((BLKC, CHUNK), jnp.int32),
        pltpu.VMEM((CHUNK, D), F32),
        pltpu.VMEM((CHUNK, D), F32),
        pltpu.VMEM_SHARED((NPAD, D), F32),
        pltpu.SemaphoreType.DMA,
        pltpu.SemaphoreType.DMA,
        pltpu.SemaphoreType.DMA,
        pltpu.SemaphoreType.DMA,
    ],
)(_scat_body)


def _pre_body(degt_ref, x_ref, w_ref, dinv_ref, hp_ref):
    deg = degt_ref[:, 0:1] + degt_ref[:, 1:2] + 1.0
    dinv = lax.rsqrt(deg)
    dinv_ref[...] = dinv
    hp_ref[...] = dinv * jnp.dot(
        x_ref[...], w_ref[...], preferred_element_type=F32
    )


def _mid_body(sp_ref, hp_ref, dinv_ref, b_ref, w_ref, o_ref):
    dinv = dinv_ref[...]
    x2 = dinv * (sp_ref[0] + sp_ref[1] + hp_ref[...]) + b_ref[...]
    o_ref[...] = dinv * jnp.dot(x2, w_ref[...], preferred_element_type=F32)


def _post_body(sp_ref, hp_ref, dinv_ref, b_ref, o_ref):
    o_ref[...] = (
        dinv_ref[...] * (sp_ref[0] + sp_ref[1] + hp_ref[...]) + b_ref[...]
    )


_GRID = (N // ROWS_BLK,)
_spec_rows = pl.BlockSpec((ROWS_BLK, D), lambda i: (i, 0))
_spec_degt = pl.BlockSpec((ROWS_BLK, NC), lambda i: (i, 0))
_spec_dinv = pl.BlockSpec((ROWS_BLK, 1), lambda i: (i, 0))
_spec_w = pl.BlockSpec((D, D), lambda i: (0, 0))
_spec_b = pl.BlockSpec((D,), lambda i: (0,))
_spec_sp = pl.BlockSpec((NC, ROWS_BLK, D), lambda i: (0, i, 0))


def _pre_call(degt, x, w):
    return pl.pallas_call(
        _pre_body,
        grid=_GRID,
        in_specs=[_spec_degt, _spec_rows, _spec_w],
        out_specs=[_spec_dinv, _spec_rows],
        out_shape=[
            jax.ShapeDtypeStruct((N, 1), F32),
            jax.ShapeDtypeStruct((N, D), F32),
        ],
    )(degt, x, w)


def _mid_call(sp, hp, dinv, b, w):
    return pl.pallas_call(
        _mid_body,
        grid=_GRID,
        in_specs=[_spec_sp, _spec_rows, _spec_dinv, _spec_b, _spec_w],
        out_specs=_spec_rows,
        out_shape=jax.ShapeDtypeStruct((N, D), F32),
    )(sp, hp, dinv, b, w)


def _post_call(sp, hp, dinv, b):
    return pl.pallas_call(
        _post_body,
        grid=_GRID,
        in_specs=[_spec_sp, _spec_rows, _spec_dinv, _spec_b],
        out_specs=_spec_rows,
        out_shape=jax.ShapeDtypeStruct((N, D), F32),
    )(sp, hp, dinv, b)


@jax.jit
def kernel(x, edge_index, W1, b1, W2, b2, W3, b3):
    ei = edge_index.astype(jnp.int32)
    npad = EPAD - E
    pad_src = jnp.arange(npad, dtype=jnp.int32) % N
    src_t = jnp.concatenate([ei[0], pad_src]).reshape(NW, CPT, CHUNK)
    pad_dst = N + (jnp.arange(npad, dtype=jnp.int32) % (NPAD - N))
    dst_t = jnp.concatenate([ei[1], pad_dst]).reshape(NW, CPT, CHUNK)
    zeros1 = jnp.zeros((NPAD,), F32)
    zeros2 = jnp.zeros((CHUNK, D), F32)

    degt = _deg_call(dst_t, zeros1).reshape(NC, NPAD)[:, :N].T

    dinv, h1p = _pre_call(degt, x, W1)
    s1 = _scat_call(h1p, src_t, dst_t, zeros2)
    h2p = _mid_call(s1, h1p, dinv, b1, W2)
    s2 = _scat_call(h2p, src_t, dst_t, zeros2)
    h3p = _mid_call(s2, h2p, dinv, b2, W3)
    s3 = _scat_call(h3p, src_t, dst_t, zeros2)
    return _post_call(s3, h3p, dinv, b3)
